# baseline (speedup 1.0000x reference)
import functools

import jax
import jax.numpy as jnp
from jax import lax
from jax.experimental import pallas as pl
from jax.experimental.pallas import tpu as pltpu
from jax.experimental.pallas import tpu_sc as plsc

B, S, D = 4, 2048, 1024
HALF = D // 2
T = B * S
NC, NS = 2, 16
NW = NC * NS
PER_W = T // NW
CHUNK = 16
NCHUNK = PER_W // CHUNK
NBUF = 3
XBUF = 4
LANES = 16
VPH = HALF // LANES

_mesh = plsc.VectorSubcoreMesh(
    core_axis_name="c", subcore_axis_name="s", num_cores=NC, num_subcores=NS)


@functools.partial(
    pl.kernel,
    out_type=jax.ShapeDtypeStruct((T, D), jnp.float32),
    mesh=_mesh,
    scratch_types=[
        pltpu.VMEM((PER_W,), jnp.int32),
        pltpu.VMEM((PER_W,), jnp.int32),
        [pltpu.VMEM((CHUNK, D), jnp.float32) for _ in range(XBUF)],
        [pltpu.VMEM((CHUNK, HALF), jnp.float32) for _ in range(NBUF)],
        [pltpu.VMEM((CHUNK, HALF), jnp.float32) for _ in range(NBUF)],
        [pltpu.SemaphoreType.DMA for _ in range(XBUF)],
        [pltpu.SemaphoreType.DMA for _ in range(NBUF)],
        [pltpu.SemaphoreType.DMA for _ in range(XBUF)],
    ],
)
def _pos_emb_add(x_hbm, idx1_hbm, idx2_hbm, pe1_hbm, pe2_hbm, out_hbm,
                 idx1_v, idx2_v, x_v, g1_v, g2_v, sem_x, sem_g, sem_out):
    wid = lax.axis_index("s") * NC + lax.axis_index("c")
    base = wid * PER_W
    pltpu.sync_copy(idx1_hbm.at[pl.ds(base, PER_W)], idx1_v)
    pltpu.sync_copy(idx2_hbm.at[pl.ds(base, PER_W)], idx2_v)

    def copies(c):
        sx = c % XBUF
        sg = c % NBUF
        off = base + c * CHUNK
        return (
            pltpu.make_async_copy(x_hbm.at[pl.ds(off, CHUNK)], x_v[sx],
                                  sem_x[sx]),
            pltpu.make_async_copy(
                pe1_hbm.at[idx1_v.at[pl.ds(c * CHUNK, CHUNK)]], g1_v[sg],
                sem_g[sg]),
            pltpu.make_async_copy(
                pe2_hbm.at[idx2_v.at[pl.ds(c * CHUNK, CHUNK)]], g2_v[sg],
                sem_g[sg]),
        )

    def issue_in(c):
        for cp in copies(c):
            cp.start()

    def issue_x(c):
        copies(c)[0].start()

    def issue_g(c):
        for cp in copies(c)[1:]:
            cp.start()

    issue_in(0)
    issue_in(1)
    issue_x(2)

    for c in range(NCHUNK):
        sx = c % XBUF
        off = base + c * CHUNK
        for cp in copies(c):
            cp.wait()

        def add_row(k, _):
            sg = c % NBUF
            for j in range(VPH):
                plsc.addupdate(x_v[sx].at[k, pl.ds(j * LANES, LANES)],
                               g1_v[sg][k, pl.ds(j * LANES, LANES)])
                plsc.addupdate(x_v[sx].at[k, pl.ds(HALF + j * LANES, LANES)],
                               g2_v[sg][k, pl.ds(j * LANES, LANES)])
            return _

        lax.fori_loop(0, CHUNK, add_row, 0)
        pltpu.async_copy(x_v[sx], out_hbm.at[pl.ds(off, CHUNK)], sem_out[sx])

        if c + 2 < NCHUNK:
            issue_g(c + 2)
        if c + 3 < NCHUNK:
            if c >= 1:
                sp = (c - 1) % XBUF
                offp = base + (c - 1) * CHUNK
                pltpu.make_async_copy(
                    x_v[sp], out_hbm.at[pl.ds(offp, CHUNK)],
                    sem_out[sp]).wait()
            issue_x(c + 3)

    for c in range(NCHUNK - XBUF, NCHUNK):
        sx = c % XBUF
        off = base + c * CHUNK
        pltpu.make_async_copy(x_v[sx], out_hbm.at[pl.ds(off, CHUNK)],
                              sem_out[sx]).wait()


def kernel(inputs, positions, pe1, pe2):
    pos = positions.astype(jnp.int32)
    idx1 = pos[:, :, 0].reshape(T)
    idx2 = pos[:, :, 1].reshape(T)
    out = _pos_emb_add(inputs.reshape(T, D), idx1, idx2, pe1, pe2)
    return out.reshape(B, S, D)
